# Initial kernel scaffold; baseline (speedup 1.0000x reference)
import jax
import jax.numpy as jnp
from jax.experimental import pallas as pl


def kernel(Graph, norm_h, norm_L, norm_W, norm_P, norm_N, T, W0, We0, al0, ar0, ae0, b0, W1, We1, al1, ar1, ae1, b1, W2, We2, al2, ar2, ae2, b2):
    raise NotImplementedError("write your pallas kernel here")


# speedup vs baseline: 1651.4966x; 1651.4966x over previous
import jax
import jax.numpy as jnp
from jax.experimental import pallas as pl

_J = 100
_M = 28
_NN = _J + _M
_H = 5
_EMBED = 64
_DIMS = (16, 64, _EMBED)


def _layer(x, m, ef, W, alm, arm, aux, out):
    ft = jnp.dot(x, W, preferred_element_type=jnp.float32)
    el = jnp.dot(ft, alm, preferred_element_type=jnp.float32)
    er_t = jax.lax.dot_general(
        arm, ft, (((0,), (1,)), ((), ())),
        preferred_element_type=jnp.float32)
    acc = jnp.zeros((_NN, out), jnp.float32)
    for h in range(_H):
        ce_h = aux[2:3, h * out:h * out + 1]
        e_h = el[:, h:h + 1] + er_t[h:h + 1, :] + ef * ce_h
        e_h = jnp.where(e_h >= 0, e_h, 0.2 * e_h)
        e_m = jnp.where(m, e_h, -3.4e38)
        emax = jnp.max(e_m, axis=0, keepdims=True)
        emax = jnp.where(emax > -1e37, emax, 0.0)
        ex = jnp.where(m, jnp.exp(e_h - emax), 0.0)
        den = jnp.sum(ex, axis=0, keepdims=True)
        a = ex / jnp.where(den > 0, den, 1.0)
        ft_h = ft[:, h * out:(h + 1) * out]
        rst = jax.lax.dot_general(
            a, ft_h, (((0,), (0,)), ((), ())),
            preferred_element_type=jnp.float32)
        we_b = jnp.broadcast_to(aux[0:1, h * out:(h + 1) * out], (_NN, out))
        rst = rst + jax.lax.dot_general(
            a * ef, we_b, (((0,), (0,)), ((), ())),
            preferred_element_type=jnp.float32)
        rst = rst + aux[1:2, h * out:(h + 1) * out]
        acc = acc + jnp.where(rst >= 0, rst, 0.01 * rst)
    return acc * (1.0 / _H)


def _body(sq_ref, ef_ref, x_ref,
          W0_ref, alm0_ref, arm0_ref, aux0_ref,
          W1_ref, alm1_ref, arm1_ref, aux1_ref,
          W2_ref, alm2_ref, arm2_ref, aux2_ref,
          out_ref):
    m = sq_ref[0] != 0.0
    ef = ef_ref[0]
    x = x_ref[0]
    x = _layer(x, m, ef, W0_ref[...], alm0_ref[...], arm0_ref[...],
               aux0_ref[...], _DIMS[0])
    x = _layer(x, m, ef, W1_ref[...], alm1_ref[...], arm1_ref[...],
               aux1_ref[...], _DIMS[1])
    x = _layer(x, m, ef, W2_ref[...], alm2_ref[...], arm2_ref[...],
               aux2_ref[...], _DIMS[2])
    out_ref[0] = x


def _blockdiag(a, out):
    eye = jnp.eye(_H, dtype=a.dtype)
    return (a[:, :, None] * eye[:, None, :]).reshape(_H * out, _H)


def _aux(We, b, ae, out):
    We2 = We.reshape(_H, out)
    ce = (We2 * ae).sum(-1)
    ce_rep = jnp.repeat(ce, out)
    return jnp.stack([We.reshape(-1), b, ce_rep], axis=0)


def kernel(Graph, norm_h, norm_L, norm_W, norm_P, norm_N, T,
           W0, We0, al0, ar0, ae0, b0,
           W1, We1, al1, ar1, ae1, b1,
           W2, We2, al2, ar2, ae2, b2):
    bs = Graph.shape[0]
    sq = jnp.zeros((bs, _NN, _NN), jnp.float32).at[:, :_J, :].set(Graph)
    Tm = jnp.zeros((bs, _NN, _NN), jnp.float32).at[:, :_J, :_J].set(T)

    jobList = jnp.stack((norm_h, norm_L), axis=-1)
    jobID = jnp.broadcast_to(
        jnp.arange(1, _J + 1, dtype=jnp.float32)[None, :, None], (bs, _J, 1))
    jw = jnp.zeros((bs, _J, 1), jnp.float32)
    jobF = jnp.concatenate((jobList, jw, jw, jw, jobID, jw), axis=-1)
    WPN = jnp.broadcast_to(
        jnp.concatenate((norm_W, norm_P, norm_N), axis=1)[:, None, :],
        (bs, _M, 3))
    mID = jnp.broadcast_to(
        jnp.arange(1, _M + 1, dtype=jnp.float32)[None, :, None], (bs, _M, 1))
    mw = jnp.zeros((bs, _M, 1), jnp.float32)
    mF = jnp.concatenate((mw, mw, WPN, mw, mID), axis=-1)
    nodeF = jnp.concatenate((jobF, mF), axis=1)

    alm0_, arm0_ = _blockdiag(al0, _DIMS[0]), _blockdiag(ar0, _DIMS[0])
    alm1_, arm1_ = _blockdiag(al1, _DIMS[1]), _blockdiag(ar1, _DIMS[1])
    alm2_, arm2_ = _blockdiag(al2, _DIMS[2]), _blockdiag(ar2, _DIMS[2])
    aux0_ = _aux(We0, b0, ae0, _DIMS[0])
    aux1_ = _aux(We1, b1, ae1, _DIMS[1])
    aux2_ = _aux(We2, b2, ae2, _DIMS[2])

    def bspec(shape3):
        return pl.BlockSpec((1,) + shape3[1:], lambda b: (b, 0, 0))

    def wspec(arr):
        return pl.BlockSpec(arr.shape, lambda b: (0,) * arr.ndim)

    out = pl.pallas_call(
        _body,
        grid=(bs,),
        in_specs=[
            bspec(sq.shape), bspec(Tm.shape), bspec(nodeF.shape),
            wspec(W0), wspec(alm0_), wspec(arm0_), wspec(aux0_),
            wspec(W1), wspec(alm1_), wspec(arm1_), wspec(aux1_),
            wspec(W2), wspec(alm2_), wspec(arm2_), wspec(aux2_),
        ],
        out_specs=pl.BlockSpec((1, _NN, _EMBED), lambda b: (b, 0, 0)),
        out_shape=jax.ShapeDtypeStruct((bs, _NN, _EMBED), jnp.float32),
    )(sq, Tm, nodeF,
      W0, alm0_, arm0_, aux0_,
      W1, alm1_, arm1_, aux1_,
      W2, alm2_, arm2_, aux2_)
    return out


# speedup vs baseline: 1652.6883x; 1.0007x over previous
import jax
import jax.numpy as jnp
from jax.experimental import pallas as pl
from jax.experimental.pallas import tpu as pltpu

_J = 100
_M = 28
_NN = _J + _M
_H = 5
_EMBED = 64
_DIMS = (16, 64, _EMBED)


def _layer(x, m, ef, W, alm, arm, aux, out):
    ft = jnp.dot(x, W, preferred_element_type=jnp.float32)
    el = jnp.dot(ft, alm, preferred_element_type=jnp.float32)
    er_t = jax.lax.dot_general(
        arm, ft, (((0,), (1,)), ((), ())),
        preferred_element_type=jnp.float32)
    acc = jnp.zeros((_NN, out), jnp.float32)
    for h in range(_H):
        ce_h = aux[2:3, h * out:h * out + 1]
        e_h = el[:, h:h + 1] + er_t[h:h + 1, :] + ef * ce_h
        e_h = jnp.where(e_h >= 0, e_h, 0.2 * e_h)
        e_m = jnp.where(m, e_h, -3.4e38)
        emax = jnp.max(e_m, axis=0, keepdims=True)
        emax = jnp.where(emax > -1e37, emax, 0.0)
        ex = jnp.where(m, jnp.exp(e_h - emax), 0.0)
        den = jnp.sum(ex, axis=0, keepdims=True)
        a = ex / jnp.where(den > 0, den, 1.0)
        ft_h = ft[:, h * out:(h + 1) * out]
        rst = jax.lax.dot_general(
            a, ft_h, (((0,), (0,)), ((), ())),
            preferred_element_type=jnp.float32)
        we_b = jnp.broadcast_to(aux[0:1, h * out:(h + 1) * out], (_NN, out))
        rst = rst + jax.lax.dot_general(
            a * ef, we_b, (((0,), (0,)), ((), ())),
            preferred_element_type=jnp.float32)
        rst = rst + aux[1:2, h * out:(h + 1) * out]
        acc = acc + jnp.where(rst >= 0, rst, 0.01 * rst)
    return acc * (1.0 / _H)


def _body(sq_ref, ef_ref, x_ref,
          W0_ref, alm0_ref, arm0_ref, aux0_ref,
          W1_ref, alm1_ref, arm1_ref, aux1_ref,
          W2_ref, alm2_ref, arm2_ref, aux2_ref,
          out_ref):
    m = sq_ref[0] != 0.0
    ef = ef_ref[0]
    x = x_ref[0]
    x = _layer(x, m, ef, W0_ref[...], alm0_ref[...], arm0_ref[...],
               aux0_ref[...], _DIMS[0])
    x = _layer(x, m, ef, W1_ref[...], alm1_ref[...], arm1_ref[...],
               aux1_ref[...], _DIMS[1])
    x = _layer(x, m, ef, W2_ref[...], alm2_ref[...], arm2_ref[...],
               aux2_ref[...], _DIMS[2])
    out_ref[0] = x


def _blockdiag(a, out):
    eye = jnp.eye(_H, dtype=a.dtype)
    return (a[:, :, None] * eye[:, None, :]).reshape(_H * out, _H)


def _aux(We, b, ae, out):
    We2 = We.reshape(_H, out)
    ce = (We2 * ae).sum(-1)
    ce_rep = jnp.repeat(ce, out)
    return jnp.stack([We.reshape(-1), b, ce_rep], axis=0)


def kernel(Graph, norm_h, norm_L, norm_W, norm_P, norm_N, T,
           W0, We0, al0, ar0, ae0, b0,
           W1, We1, al1, ar1, ae1, b1,
           W2, We2, al2, ar2, ae2, b2):
    bs = Graph.shape[0]
    sq = jnp.zeros((bs, _NN, _NN), jnp.float32).at[:, :_J, :].set(Graph)
    Tm = jnp.zeros((bs, _NN, _NN), jnp.float32).at[:, :_J, :_J].set(T)

    jobList = jnp.stack((norm_h, norm_L), axis=-1)
    jobID = jnp.broadcast_to(
        jnp.arange(1, _J + 1, dtype=jnp.float32)[None, :, None], (bs, _J, 1))
    jw = jnp.zeros((bs, _J, 1), jnp.float32)
    jobF = jnp.concatenate((jobList, jw, jw, jw, jobID, jw), axis=-1)
    WPN = jnp.broadcast_to(
        jnp.concatenate((norm_W, norm_P, norm_N), axis=1)[:, None, :],
        (bs, _M, 3))
    mID = jnp.broadcast_to(
        jnp.arange(1, _M + 1, dtype=jnp.float32)[None, :, None], (bs, _M, 1))
    mw = jnp.zeros((bs, _M, 1), jnp.float32)
    mF = jnp.concatenate((mw, mw, WPN, mw, mID), axis=-1)
    nodeF = jnp.concatenate((jobF, mF), axis=1)

    alm0_, arm0_ = _blockdiag(al0, _DIMS[0]), _blockdiag(ar0, _DIMS[0])
    alm1_, arm1_ = _blockdiag(al1, _DIMS[1]), _blockdiag(ar1, _DIMS[1])
    alm2_, arm2_ = _blockdiag(al2, _DIMS[2]), _blockdiag(ar2, _DIMS[2])
    aux0_ = _aux(We0, b0, ae0, _DIMS[0])
    aux1_ = _aux(We1, b1, ae1, _DIMS[1])
    aux2_ = _aux(We2, b2, ae2, _DIMS[2])

    def bspec(shape3):
        return pl.BlockSpec((1,) + shape3[1:], lambda b: (b, 0, 0))

    def wspec(arr):
        return pl.BlockSpec(arr.shape, lambda b: (0,) * arr.ndim)

    out = pl.pallas_call(
        _body,
        grid=(bs,),
        in_specs=[
            bspec(sq.shape), bspec(Tm.shape), bspec(nodeF.shape),
            wspec(W0), wspec(alm0_), wspec(arm0_), wspec(aux0_),
            wspec(W1), wspec(alm1_), wspec(arm1_), wspec(aux1_),
            wspec(W2), wspec(alm2_), wspec(arm2_), wspec(aux2_),
        ],
        out_specs=pl.BlockSpec((1, _NN, _EMBED), lambda b: (b, 0, 0)),
        out_shape=jax.ShapeDtypeStruct((bs, _NN, _EMBED), jnp.float32),
        compiler_params=pltpu.CompilerParams(
            dimension_semantics=("parallel",)),
    )(sq, Tm, nodeF,
      W0, alm0_, arm0_, aux0_,
      W1, alm1_, arm1_, aux1_,
      W2, alm2_, arm2_, aux2_)
    return out


# speedup vs baseline: 1730.1316x; 1.0469x over previous
import jax
import jax.numpy as jnp
from jax.experimental import pallas as pl
from jax.experimental.pallas import tpu as pltpu

_J = 100
_M = 28
_NN = _J + _M
_H = 5
_EMBED = 64
_DIMS = (16, 64, _EMBED)
_WID = _H * _NN


def _layer(x, m5, ef5, W, almS, arm, aux, ces, out):
    ft = jnp.dot(x, W, preferred_element_type=jnp.float32)
    el_part = jnp.dot(ft, almS, preferred_element_type=jnp.float32)
    er_t = jax.lax.dot_general(
        arm, ft, (((0,), (1,)), ((), ())),
        preferred_element_type=jnp.float32)
    er_s = jnp.concatenate([er_t[h:h + 1, :] for h in range(_H)], axis=1)
    e = el_part + er_s + ef5 * ces
    e = jnp.maximum(e, 0.2 * e)
    e_m = jnp.where(m5, e, -3.4e38)
    emax = jnp.max(e_m, axis=0, keepdims=True)
    emax = jnp.where(emax > -1e37, emax, 0.0)
    ex = jnp.exp(e_m - emax)
    den = jnp.sum(ex, axis=0, keepdims=True)
    a = ex * (1.0 / jnp.where(den > 0, den, 1.0))
    aef = a * ef5
    acc = jnp.zeros((_NN, out), jnp.float32)
    for h in range(_H):
        sl = slice(h * _NN, (h + 1) * _NN)
        ft_h = ft[:, h * out:(h + 1) * out]
        rst = jax.lax.dot_general(
            a[:, sl], ft_h, (((0,), (0,)), ((), ())),
            preferred_element_type=jnp.float32)
        we_b = jnp.broadcast_to(aux[0:1, h * out:(h + 1) * out], (_NN, out))
        rst = rst + jax.lax.dot_general(
            aef[:, sl], we_b, (((0,), (0,)), ((), ())),
            preferred_element_type=jnp.float32)
        rst = rst + aux[1:2, h * out:(h + 1) * out]
        acc = acc + jnp.maximum(rst, 0.01 * rst)
    return acc * (1.0 / _H)


def _body(sq_ref, ef_ref, x_ref,
          W0_ref, almS0_ref, arm0_ref, aux0_ref, ces0_ref,
          W1_ref, almS1_ref, arm1_ref, aux1_ref, ces1_ref,
          W2_ref, almS2_ref, arm2_ref, aux2_ref, ces2_ref,
          out_ref):
    m = sq_ref[0] != 0.0
    ef = ef_ref[0]
    m5 = jnp.concatenate([m] * _H, axis=1)
    ef5 = jnp.concatenate([ef] * _H, axis=1)
    x = x_ref[0]
    x = _layer(x, m5, ef5, W0_ref[...], almS0_ref[...], arm0_ref[...],
               aux0_ref[...], ces0_ref[...], _DIMS[0])
    x = _layer(x, m5, ef5, W1_ref[...], almS1_ref[...], arm1_ref[...],
               aux1_ref[...], ces1_ref[...], _DIMS[1])
    x = _layer(x, m5, ef5, W2_ref[...], almS2_ref[...], arm2_ref[...],
               aux2_ref[...], ces2_ref[...], _DIMS[2])
    out_ref[0] = x


def _almS(a, out):
    eye = jnp.eye(_H, dtype=a.dtype)
    alm = (a[:, :, None] * eye[:, None, :]).reshape(_H * out, _H)
    return jnp.repeat(alm, _NN, axis=1)


def _arm(a, out):
    eye = jnp.eye(_H, dtype=a.dtype)
    return (a[:, :, None] * eye[:, None, :]).reshape(_H * out, _H)


def _aux(We, b):
    return jnp.stack([We.reshape(-1), b], axis=0)


def _ces(We, ae, out):
    ce = (We.reshape(_H, out) * ae).sum(-1)
    return jnp.repeat(ce, _NN)[None, :]


def kernel(Graph, norm_h, norm_L, norm_W, norm_P, norm_N, T,
           W0, We0, al0, ar0, ae0, b0,
           W1, We1, al1, ar1, ae1, b1,
           W2, We2, al2, ar2, ae2, b2):
    bs = Graph.shape[0]
    sq = jnp.zeros((bs, _NN, _NN), jnp.float32).at[:, :_J, :].set(Graph)
    Tm = jnp.zeros((bs, _NN, _NN), jnp.float32).at[:, :_J, :_J].set(T)

    jobList = jnp.stack((norm_h, norm_L), axis=-1)
    jobID = jnp.broadcast_to(
        jnp.arange(1, _J + 1, dtype=jnp.float32)[None, :, None], (bs, _J, 1))
    jw = jnp.zeros((bs, _J, 1), jnp.float32)
    jobF = jnp.concatenate((jobList, jw, jw, jw, jobID, jw), axis=-1)
    WPN = jnp.broadcast_to(
        jnp.concatenate((norm_W, norm_P, norm_N), axis=1)[:, None, :],
        (bs, _M, 3))
    mID = jnp.broadcast_to(
        jnp.arange(1, _M + 1, dtype=jnp.float32)[None, :, None], (bs, _M, 1))
    mw = jnp.zeros((bs, _M, 1), jnp.float32)
    mF = jnp.concatenate((mw, mw, WPN, mw, mID), axis=-1)
    nodeF = jnp.concatenate((jobF, mF), axis=1)

    almS0_, arm0_ = _almS(al0, _DIMS[0]), _arm(ar0, _DIMS[0])
    almS1_, arm1_ = _almS(al1, _DIMS[1]), _arm(ar1, _DIMS[1])
    almS2_, arm2_ = _almS(al2, _DIMS[2]), _arm(ar2, _DIMS[2])
    aux0_, ces0_ = _aux(We0, b0), _ces(We0, ae0, _DIMS[0])
    aux1_, ces1_ = _aux(We1, b1), _ces(We1, ae1, _DIMS[1])
    aux2_, ces2_ = _aux(We2, b2), _ces(We2, ae2, _DIMS[2])

    def bspec(shape3):
        return pl.BlockSpec((1,) + shape3[1:], lambda b: (b, 0, 0))

    def wspec(arr):
        return pl.BlockSpec(arr.shape, lambda b: (0,) * arr.ndim)

    out = pl.pallas_call(
        _body,
        grid=(bs,),
        in_specs=[
            bspec(sq.shape), bspec(Tm.shape), bspec(nodeF.shape),
            wspec(W0), wspec(almS0_), wspec(arm0_), wspec(aux0_), wspec(ces0_),
            wspec(W1), wspec(almS1_), wspec(arm1_), wspec(aux1_), wspec(ces1_),
            wspec(W2), wspec(almS2_), wspec(arm2_), wspec(aux2_), wspec(ces2_),
        ],
        out_specs=pl.BlockSpec((1, _NN, _EMBED), lambda b: (b, 0, 0)),
        out_shape=jax.ShapeDtypeStruct((bs, _NN, _EMBED), jnp.float32),
        compiler_params=pltpu.CompilerParams(
            dimension_semantics=("parallel",)),
    )(sq, Tm, nodeF,
      W0, almS0_, arm0_, aux0_, ces0_,
      W1, almS1_, arm1_, aux1_, ces1_,
      W2, almS2_, arm2_, aux2_, ces2_)
    return out
